# baseline (speedup 1.0000x reference)
import functools

import jax
import jax.numpy as jnp
from jax.experimental import pallas as pl
from jax.experimental.pallas import tpu as pltpu


def _proj_body(s1_ref, s2_ref, w_ref, h_ref, *, nh):
    w = w_ref[...]
    h_ref[:, :nh] = jnp.dot(s1_ref[...], w, preferred_element_type=jnp.float32)
    h_ref[:, nh:] = jnp.dot(s2_ref[...], w, preferred_element_type=jnp.float32)


def _gcn_body(adj_ref, hp_ref, b_ref, a_ref, h_ref, csum_ref, *, nh):
    part = jnp.dot(adj_ref[...], hp_ref[...], preferred_element_type=jnp.float32)
    g = part + b_ref[...]
    h = jnp.where(g > 0, g, a_ref[...] * g)
    h_ref[...] = h
    csum_ref[...] = jnp.sum(h[:, :nh], axis=0, keepdims=True)[None]


def _disc_body(h_ref, csums_ref, wd_ref, bd_ref, o1_ref, o2_ref, *, n_nodes, nh):
    csum = jnp.sum(csums_ref[...], axis=0)
    c = jax.nn.sigmoid(csum * (1.0 / n_nodes))
    wd = wd_ref[...]
    t1 = jnp.dot(h_ref[:, :nh], wd, preferred_element_type=jnp.float32)
    t2 = jnp.dot(h_ref[:, nh:], wd, preferred_element_type=jnp.float32)
    o1_ref[...] = jnp.sum(t1 * c, axis=-1, keepdims=True) + bd_ref[...]
    o2_ref[...] = jnp.sum(t2 * c, axis=-1, keepdims=True) + bd_ref[...]


def kernel(seq1, seq2, adj, sparse, W_fc, b_gcn, a_prelu, W_disc, b_disc):
    n = seq1.shape[1]
    nin = W_fc.shape[0]
    nh = W_fc.shape[1]
    s1 = seq1.reshape(n, nin)
    s2 = seq2.reshape(n, nin)
    a2 = adj.reshape(n, n)
    b2 = jnp.concatenate([b_gcn, b_gcn]).reshape(1, 2 * nh)
    a_p = jnp.asarray(a_prelu, jnp.float32).reshape(1, 1)
    bd = jnp.asarray(b_disc, jnp.float32).reshape(1, 1)

    bp = 2000
    bm = 200
    n_i = n // bm
    bd_rows = 2000

    hp = pl.pallas_call(
        functools.partial(_proj_body, nh=nh),
        grid=(n // bp,),
        in_specs=[
            pl.BlockSpec((bp, nin), lambda p: (p, 0)),
            pl.BlockSpec((bp, nin), lambda p: (p, 0)),
            pl.BlockSpec((nin, nh), lambda p: (0, 0)),
        ],
        out_specs=pl.BlockSpec((bp, 2 * nh), lambda p: (p, 0)),
        out_shape=jax.ShapeDtypeStruct((n, 2 * nh), jnp.float32),
        compiler_params=pltpu.CompilerParams(
            dimension_semantics=("parallel",),
        ),
    )(s1, s2, W_fc)

    h, csums = pl.pallas_call(
        functools.partial(_gcn_body, nh=nh),
        grid=(n_i,),
        in_specs=[
            pl.BlockSpec((bm, n), lambda i: (i, 0)),
            pl.BlockSpec((n, 2 * nh), lambda i: (0, 0)),
            pl.BlockSpec((1, 2 * nh), lambda i: (0, 0)),
            pl.BlockSpec((1, 1), lambda i: (0, 0)),
        ],
        out_specs=[
            pl.BlockSpec((bm, 2 * nh), lambda i: (i, 0)),
            pl.BlockSpec((1, 1, nh), lambda i: (i, 0, 0)),
        ],
        out_shape=[
            jax.ShapeDtypeStruct((n, 2 * nh), jnp.float32),
            jax.ShapeDtypeStruct((n_i, 1, nh), jnp.float32),
        ],
        compiler_params=pltpu.CompilerParams(
            dimension_semantics=("parallel",),
        ),
    )(a2, hp, b2, a_p)

    sc1, sc2 = pl.pallas_call(
        functools.partial(_disc_body, n_nodes=float(n), nh=nh),
        grid=(n // bd_rows,),
        in_specs=[
            pl.BlockSpec((bd_rows, 2 * nh), lambda d: (d, 0)),
            pl.BlockSpec((n_i, 1, nh), lambda d: (0, 0, 0)),
            pl.BlockSpec((nh, nh), lambda d: (0, 0)),
            pl.BlockSpec((1, 1), lambda d: (0, 0)),
        ],
        out_specs=[
            pl.BlockSpec((bd_rows, 1), lambda d: (d, 0)),
            pl.BlockSpec((bd_rows, 1), lambda d: (d, 0)),
        ],
        out_shape=[
            jax.ShapeDtypeStruct((n, 1), jnp.float32),
            jax.ShapeDtypeStruct((n, 1), jnp.float32),
        ],
        compiler_params=pltpu.CompilerParams(
            dimension_semantics=("parallel",),
        ),
    )(h, csums, W_disc, bd)

    return jnp.concatenate([sc1.reshape(1, n), sc2.reshape(1, n)], axis=1)


# speedup vs baseline: 1.0970x; 1.0970x over previous
import functools

import jax
import jax.numpy as jnp
from jax.experimental import pallas as pl
from jax.experimental.pallas import tpu as pltpu


def _dgi_body(adj_ref, s1_ref, s2_ref, wfc_ref, b_ref, a_ref, wd_ref, bd_ref,
              sc1_ref, sc2_ref, hp_s, h_s, csum_s, *, n_i, bm, nh, n):
    i = pl.program_id(0)

    @pl.when(i == 0)
    def _():
        w = wfc_ref[...]
        hp_s[:, :nh] = jnp.dot(s1_ref[...], w, preferred_element_type=jnp.float32)
        hp_s[:, nh:] = jnp.dot(s2_ref[...], w, preferred_element_type=jnp.float32)
        csum_s[...] = jnp.zeros_like(csum_s)

    part = jnp.dot(adj_ref[...], hp_s[...], preferred_element_type=jnp.float32)
    g = part + b_ref[...]
    h = jnp.where(g > 0, g, a_ref[...] * g)
    h_s[pl.ds(i * bm, bm), :] = h
    csum_s[...] += jnp.sum(h[:, :nh], axis=0, keepdims=True)

    @pl.when(i == n_i - 1)
    def _():
        c = jax.nn.sigmoid(csum_s[...] * (1.0 / n))
        wd = wd_ref[...]
        t1 = jnp.dot(h_s[:, :nh], wd, preferred_element_type=jnp.float32)
        t2 = jnp.dot(h_s[:, nh:], wd, preferred_element_type=jnp.float32)
        sc1_ref[...] = jnp.sum(t1 * c, axis=-1, keepdims=True) + bd_ref[...]
        sc2_ref[...] = jnp.sum(t2 * c, axis=-1, keepdims=True) + bd_ref[...]


def kernel(seq1, seq2, adj, sparse, W_fc, b_gcn, a_prelu, W_disc, b_disc):
    n = seq1.shape[1]
    nin = W_fc.shape[0]
    nh = W_fc.shape[1]
    s1 = seq1.reshape(n, nin)
    s2 = seq2.reshape(n, nin)
    a2 = adj.reshape(n, n)
    b2 = jnp.concatenate([b_gcn, b_gcn]).reshape(1, 2 * nh)
    a_p = jnp.asarray(a_prelu, jnp.float32).reshape(1, 1)
    bd = jnp.asarray(b_disc, jnp.float32).reshape(1, 1)

    bm = 200
    n_i = n // bm

    sc1, sc2 = pl.pallas_call(
        functools.partial(_dgi_body, n_i=n_i, bm=bm, nh=nh, n=float(n)),
        grid=(n_i,),
        in_specs=[
            pl.BlockSpec((bm, n), lambda i: (i, 0)),
            pl.BlockSpec((n, nin), lambda i: (0, 0)),
            pl.BlockSpec((n, nin), lambda i: (0, 0)),
            pl.BlockSpec((nin, nh), lambda i: (0, 0)),
            pl.BlockSpec((1, 2 * nh), lambda i: (0, 0)),
            pl.BlockSpec((1, 1), lambda i: (0, 0)),
            pl.BlockSpec((nh, nh), lambda i: (0, 0)),
            pl.BlockSpec((1, 1), lambda i: (0, 0)),
        ],
        out_specs=[
            pl.BlockSpec((n, 1), lambda i: (0, 0)),
            pl.BlockSpec((n, 1), lambda i: (0, 0)),
        ],
        out_shape=[
            jax.ShapeDtypeStruct((n, 1), jnp.float32),
            jax.ShapeDtypeStruct((n, 1), jnp.float32),
        ],
        scratch_shapes=[
            pltpu.VMEM((n, 2 * nh), jnp.float32),
            pltpu.VMEM((n, 2 * nh), jnp.float32),
            pltpu.VMEM((1, nh), jnp.float32),
        ],
        compiler_params=pltpu.CompilerParams(
            dimension_semantics=("arbitrary",),
        ),
    )(a2, s1, s2, W_fc, b2, a_p, W_disc, bd)

    return jnp.concatenate([sc1.reshape(1, n), sc2.reshape(1, n)], axis=1)
